# Initial kernel scaffold; baseline (speedup 1.0000x reference)
import jax
import jax.numpy as jnp
from jax.experimental import pallas as pl


def kernel(hidden, cu_seqlens, termWeight, W_in, b_in, W_score, b_score):
    raise NotImplementedError("write your pallas kernel here")


# speedup vs baseline: 13.2053x; 13.2053x over previous
import functools

import jax
import jax.numpy as jnp
from jax.experimental import pallas as pl
from jax.experimental.pallas import tpu as pltpu

MAX_SPAN = 8
NEG = -1e30
BLK = 1024


def _span_body(cu_ref, hid_ref, tw_ref, win_ref, bin_ref, wsc_ref, bsc_ref,
               out_ref, *, T):
    i = pl.program_id(0)
    D = win_ref.shape[0]
    qv = jnp.dot(win_ref[:, :], tw_ref[:, :],
                 preferred_element_type=jnp.float32) + bin_ref[:, :]
    proj = jnp.concatenate(
        [qv, wsc_ref[:, :], jnp.zeros((D, 126), jnp.float32)], axis=1)
    hid_blk = hid_ref[pl.ds(i * BLK, BLK + MAX_SPAN), :]
    lc = jnp.dot(hid_blk, proj,
                 preferred_element_type=jnp.float32)

    l = lc[:, 0:1]
    c = lc[:, 1:2]

    Lw = jnp.concatenate([l[w:w + BLK, :] for w in range(MAX_SPAN)], axis=1)
    Cw = jnp.concatenate([c[w:w + BLK, :] for w in range(MAX_SPAN)], axis=1)

    pos = i * BLK + jax.lax.broadcasted_iota(jnp.int32, (BLK, 1), 0)
    seq_end = jnp.full((BLK, 1), T, jnp.int32)
    for j in range(1, MAX_SPAN + 1):
        b = cu_ref[j]
        seq_end = jnp.minimum(seq_end, jnp.where(b > pos, b, T))
    rem = seq_end - pos

    offs = jax.lax.broadcasted_iota(jnp.int32, (1, MAX_SPAN), 1)
    bsc = bsc_ref[0, 0]
    cols = []
    for s in range(MAX_SPAN):
        span_len = jnp.minimum(s + 1, rem)
        z = jnp.where(offs < span_len, Lw, NEG)
        m = jnp.max(z, axis=1, keepdims=True)
        e = jnp.exp(z - m)
        denom = jnp.sum(e, axis=1, keepdims=True)
        num = jnp.sum(e * Cw, axis=1, keepdims=True)
        cols.append(num / denom + bsc)
    out_ref[:, :] = jnp.concatenate(cols, axis=1)


@jax.jit
def kernel(hidden, cu_seqlens, termWeight, W_in, b_in, W_score, b_score):
    T, D = hidden.shape
    hid_pad = jnp.concatenate(
        [hidden, jnp.zeros((MAX_SPAN, D), hidden.dtype)], axis=0)
    full = lambda shape: pl.BlockSpec(shape, lambda i: (0, 0),
                                      memory_space=pltpu.VMEM)
    out = pl.pallas_call(
        functools.partial(_span_body, T=T),
        grid=(T // BLK,),
        out_shape=jax.ShapeDtypeStruct((T, MAX_SPAN), jnp.float32),
        in_specs=[
            pl.BlockSpec(memory_space=pltpu.SMEM),
            full((T + MAX_SPAN, D)),
            full((D, 1)),
            full((D, D)),
            full((D, 1)),
            full((D, 1)),
            full((1, 1)),
        ],
        out_specs=pl.BlockSpec((BLK, MAX_SPAN), lambda i: (i, 0),
                               memory_space=pltpu.VMEM),
    )(cu_seqlens, hid_pad, termWeight.reshape(D, 1), W_in,
      b_in.reshape(D, 1), W_score, b_score.reshape(1, 1))
    return out.reshape(T * MAX_SPAN, 1)


# speedup vs baseline: 19.7107x; 1.4926x over previous
import functools

import jax
import jax.numpy as jnp
from jax.experimental import pallas as pl
from jax.experimental.pallas import tpu as pltpu

MAX_SPAN = 8
NEG = -1e30


def _span_body(cu_ref, hidT_ref, twT_ref, winT_ref, binT_ref, wscT_ref,
               bsc_ref, out_ref, *, T):
    D = winT_ref.shape[0]
    qvT = jnp.dot(twT_ref[:, :], winT_ref[:, :],
                  preferred_element_type=jnp.float32) + binT_ref[:, :]
    projT = jnp.concatenate(
        [qvT, wscT_ref[:, :], jnp.zeros((MAX_SPAN - 2, D), jnp.float32)],
        axis=0)
    lcT = jnp.dot(projT, hidT_ref[:, :],
                  preferred_element_type=jnp.float32)

    lT = lcT[0:1, :]
    cT = lcT[1:2, :]

    LwT = jnp.concatenate([lT[:, w:w + T] for w in range(MAX_SPAN)], axis=0)
    CwT = jnp.concatenate([cT[:, w:w + T] for w in range(MAX_SPAN)], axis=0)

    pos = jax.lax.broadcasted_iota(jnp.int32, (1, T), 1)
    seq_end = jnp.full((1, T), T, jnp.int32)
    for j in range(1, MAX_SPAN + 1):
        b = cu_ref[j]
        seq_end = jnp.minimum(seq_end, jnp.where(b > pos, b, T))
    rem = seq_end - pos

    wrow = jax.lax.broadcasted_iota(jnp.int32, (MAX_SPAN, 1), 0)
    zfull = jnp.where(wrow < rem, LwT, NEG)
    bsc = bsc_ref[0, 0]
    rows = []
    for s in range(MAX_SPAN):
        z = zfull[:s + 1]
        m = jnp.max(z, axis=0, keepdims=True)
        e = jnp.exp(z - m)
        denom = jnp.sum(e, axis=0, keepdims=True)
        num = jnp.sum(e * CwT[:s + 1], axis=0, keepdims=True)
        rows.append(num / denom + bsc)
    out_ref[:, :] = jnp.concatenate(rows, axis=0)


@jax.jit
def kernel(hidden, cu_seqlens, termWeight, W_in, b_in, W_score, b_score):
    T, D = hidden.shape
    hidT = jnp.concatenate(
        [hidden, jnp.zeros((MAX_SPAN, D), hidden.dtype)], axis=0).T
    full = lambda shape: pl.BlockSpec(shape, lambda: (0, 0),
                                      memory_space=pltpu.VMEM)
    outT = pl.pallas_call(
        functools.partial(_span_body, T=T),
        out_shape=jax.ShapeDtypeStruct((MAX_SPAN, T), jnp.float32),
        in_specs=[
            pl.BlockSpec(memory_space=pltpu.SMEM),
            full((D, T + MAX_SPAN)),
            full((1, D)),
            full((D, D)),
            full((1, D)),
            full((1, D)),
            full((1, 1)),
        ],
        out_specs=full((MAX_SPAN, T)),
    )(cu_seqlens, hidT, termWeight.reshape(1, D), W_in.T,
      b_in.reshape(1, D), W_score.reshape(1, D), b_score.reshape(1, 1))
    return outT.T.reshape(T * MAX_SPAN, 1)


# speedup vs baseline: 43.2411x; 2.1938x over previous
import functools

import jax
import jax.numpy as jnp
from jax.experimental import pallas as pl
from jax.experimental.pallas import tpu as pltpu

MAX_SPAN = 8
NEG = -1e30


def _span_body(cu_ref, hid_ref, twT_ref, winT_ref, binT_ref, wscT_ref,
               bsc_ref, out_ref, *, T):
    D = winT_ref.shape[0]
    qvT = jnp.dot(twT_ref[:, :], winT_ref[:, :],
                  preferred_element_type=jnp.float32) + binT_ref[:, :]
    projT = jnp.concatenate(
        [qvT, wscT_ref[:, :], jnp.zeros((MAX_SPAN - 2, D), jnp.float32)],
        axis=0)
    lcT = jax.lax.dot_general(
        projT, hid_ref[:, :], (((1,), (1,)), ((), ())),
        preferred_element_type=jnp.float32)

    lT = lcT[0:1, :]
    cT = lcT[1:2, :]

    LwT = jnp.concatenate([lT[:, w:w + T] for w in range(MAX_SPAN)], axis=0)
    CwT = jnp.concatenate([cT[:, w:w + T] for w in range(MAX_SPAN)], axis=0)

    pos = jax.lax.broadcasted_iota(jnp.int32, (1, T), 1)
    seq_end = jnp.full((1, T), T, jnp.int32)
    for j in range(1, MAX_SPAN + 1):
        b = cu_ref[j]
        seq_end = jnp.minimum(seq_end, jnp.where(b > pos, b, T))
    rem = seq_end - pos

    wrow = jax.lax.broadcasted_iota(jnp.int32, (MAX_SPAN, 1), 0)
    zfull = jnp.where(wrow < rem, LwT, NEG)
    bsc = bsc_ref[0, 0]
    rows = []
    for s in range(MAX_SPAN):
        z = zfull[:s + 1]
        m = jnp.max(z, axis=0, keepdims=True)
        e = jnp.exp(z - m)
        denom = jnp.sum(e, axis=0, keepdims=True)
        num = jnp.sum(e * CwT[:s + 1], axis=0, keepdims=True)
        rows.append(num / denom + bsc)
    out_ref[:, :] = jnp.concatenate(rows, axis=0)


@jax.jit
def kernel(hidden, cu_seqlens, termWeight, W_in, b_in, W_score, b_score):
    T, D = hidden.shape
    hid_pad = jnp.concatenate(
        [hidden, jnp.zeros((MAX_SPAN, D), hidden.dtype)], axis=0)
    full = lambda shape: pl.BlockSpec(shape, lambda: (0, 0),
                                      memory_space=pltpu.VMEM)
    outT = pl.pallas_call(
        functools.partial(_span_body, T=T),
        out_shape=jax.ShapeDtypeStruct((MAX_SPAN, T), jnp.float32),
        in_specs=[
            pl.BlockSpec(memory_space=pltpu.SMEM),
            full((T + MAX_SPAN, D)),
            full((1, D)),
            full((D, D)),
            full((1, D)),
            full((1, D)),
            full((1, 1)),
        ],
        out_specs=full((MAX_SPAN, T)),
    )(cu_seqlens, hid_pad, termWeight.reshape(1, D), W_in.T,
      b_in.reshape(1, D), W_score.reshape(1, D), b_score.reshape(1, 1))
    return outT.T.reshape(T * MAX_SPAN, 1)


# speedup vs baseline: 61.6281x; 1.4252x over previous
import functools

import jax
import jax.numpy as jnp
from jax.experimental import pallas as pl
from jax.experimental.pallas import tpu as pltpu

MAX_SPAN = 8
NEG = -1e30


def _span_body(cu_ref, hid_ref, twT_ref, winT_ref, binT_ref, wscT_ref,
               bsc_ref, out_ref, *, T):
    D = winT_ref.shape[0]
    qvT = jnp.dot(twT_ref[:, :], winT_ref[:, :],
                  preferred_element_type=jnp.float32) + binT_ref[:, :]
    projT = jnp.concatenate(
        [qvT, wscT_ref[:, :], jnp.zeros((MAX_SPAN - 2, D), jnp.float32)],
        axis=0)
    lcT = jax.lax.dot_general(
        projT, hid_ref[:, :], (((1,), (1,)), ((), ())),
        preferred_element_type=jnp.float32)

    lc_ext = jnp.concatenate([lcT, lcT[:, :MAX_SPAN]], axis=1)
    lT = lc_ext[0:1, :]
    cT = lc_ext[1:2, :]

    LwT = jnp.concatenate([lT[:, w:w + T] for w in range(MAX_SPAN)], axis=0)
    CwT = jnp.concatenate([cT[:, w:w + T] for w in range(MAX_SPAN)], axis=0)

    pos = jax.lax.broadcasted_iota(jnp.int32, (1, T), 1)
    seq_end = jnp.full((1, T), T, jnp.int32)
    for j in range(1, MAX_SPAN + 1):
        b = cu_ref[j]
        seq_end = jnp.minimum(seq_end, jnp.where(b > pos, b, T))
    rem = seq_end - pos

    wrow = jax.lax.broadcasted_iota(jnp.int32, (MAX_SPAN, 1), 0)
    zfull = jnp.where(wrow < rem, LwT, NEG)
    bsc = bsc_ref[0, 0]
    rows = []
    for s in range(MAX_SPAN):
        z = zfull[:s + 1]
        m = jnp.max(z, axis=0, keepdims=True)
        e = jnp.exp(z - m)
        denom = jnp.sum(e, axis=0, keepdims=True)
        num = jnp.sum(e * CwT[:s + 1], axis=0, keepdims=True)
        rows.append(num / denom + bsc)
    out_ref[:, :] = jnp.concatenate(rows, axis=0)


@jax.jit
def kernel(hidden, cu_seqlens, termWeight, W_in, b_in, W_score, b_score):
    T, D = hidden.shape
    full = lambda shape: pl.BlockSpec(shape, lambda: (0, 0),
                                      memory_space=pltpu.VMEM)
    outT = pl.pallas_call(
        functools.partial(_span_body, T=T),
        out_shape=jax.ShapeDtypeStruct((MAX_SPAN, T), jnp.float32),
        in_specs=[
            pl.BlockSpec(memory_space=pltpu.SMEM),
            full((T, D)),
            full((1, D)),
            full((D, D)),
            full((1, D)),
            full((1, D)),
            full((1, 1)),
        ],
        out_specs=full((MAX_SPAN, T)),
    )(cu_seqlens, hidden, termWeight.reshape(1, D), W_in.T,
      b_in.reshape(1, D), W_score.reshape(1, D), b_score.reshape(1, 1))
    return outT.T.reshape(T * MAX_SPAN, 1)
